# baseline (speedup 1.0000x reference)
import functools
import jax
import jax.numpy as jnp
from jax import lax
from jax.experimental import pallas as pl
from jax.experimental.pallas import tpu as pltpu
from jax.experimental.pallas import tpu_sc as plsc

VOCAB = 1000000
D = 64
LANES = 128
NC, NS = 2, 16
NW = NC * NS
CHUNK = 512
NBUF = 3

NVT = VOCAB // LANES
VT_PER_W = NVT // NW
VT_REM = NVT % NW


def _transpose_body(xt_hbm, tail_hbm, xp_hbm, buf, ob, sem_i, sem_o):
    wid = lax.axis_index("s") * NC + lax.axis_index("c")
    nblk = VT_PER_W + jnp.where(wid < VT_REM, 1, 0)
    tv0 = wid * VT_PER_W + jnp.minimum(wid, VT_REM)

    didx = []
    for cg in range(8):
        didx.append((16 * cg) % 64 + lax.iota(jnp.int32, 16))

    def do_tile(tv):
        pltpu.async_copy(
            xt_hbm.at[:, pl.ds(tv * LANES, LANES)], buf, sem_i).wait()

        def row(rr, carry):
            for cg in range(8):
                vl = 2 * rr + (1 if cg >= 4 else 0)
                v = plsc.load_gather(
                    buf, [didx[cg], jnp.full((16,), vl, jnp.int32)])
                ob[rr, pl.ds(16 * cg, 16)] = v
            return carry

        lax.fori_loop(0, LANES // 2, row, 0)
        pltpu.async_copy(
            ob, xp_hbm.at[pl.ds(tv * (LANES // 2), LANES // 2), :],
            sem_o).wait()

    def blk(i, carry):
        do_tile(tv0 + i)
        return carry

    lax.fori_loop(0, nblk, blk, 0)

    @pl.when(wid == NW - 1)
    def _tail():
        pltpu.async_copy(tail_hbm, ob.at[pl.ds(0, 32), :], sem_i).wait()
        pltpu.async_copy(
            ob.at[pl.ds(0, 32), :],
            xp_hbm.at[pl.ds(NVT * (LANES // 2), 32), :], sem_o).wait()


def _gather_body(idx_hbm, table_hbm, out_hbm, idx_v, *scratch,
                 b_per_w, nchunk):
    bufs = scratch[:NBUF]
    sem_g = scratch[NBUF:2 * NBUF]
    sem_s = scratch[2 * NBUF:3 * NBUF]

    wid = lax.axis_index("s") * NC + lax.axis_index("c")
    base = wid * b_per_w
    pltpu.sync_copy(idx_hbm.at[pl.ds(base, b_per_w)], idx_v)

    gathers = {}
    stores = {}
    store_waited = set()

    def start_gather(g):
        b = g % NBUF
        gathers[g] = pltpu.async_copy(
            table_hbm.at[idx_v.at[pl.ds(g * CHUNK, CHUNK)]], bufs[b],
            sem_g[b])

    for g in range(min(NBUF, nchunk)):
        start_gather(g)

    for g in range(nchunk):
        b = g % NBUF
        gathers[g].wait()
        stores[g] = pltpu.async_copy(
            bufs[b], out_hbm.at[pl.ds(base + g * CHUNK, CHUNK)], sem_s[b])
        t = g - 1 + NBUF
        if g >= 1 and t < nchunk:
            stores[g - 1].wait()
            store_waited.add(g - 1)
            start_gather(t)

    for g in range(nchunk):
        if g not in store_waited:
            stores[g].wait()


def kernel(indices, X):
    batch, n_fields = indices.shape
    b_total = batch * n_fields
    assert b_total % (8 * NW) == 0
    b_per_w = b_total // NW
    assert b_per_w % CHUNK == 0
    nchunk = b_per_w // CHUNK

    flat_idx = indices.reshape(b_total).astype(jnp.int32)

    mesh = plsc.VectorSubcoreMesh(core_axis_name="c", subcore_axis_name="s")

    transpose = pl.kernel(
        _transpose_body,
        mesh=mesh,
        out_type=jax.ShapeDtypeStruct((VOCAB // 2, LANES), jnp.float32),
        scratch_types=[
            pltpu.VMEM((D, LANES), jnp.float32),
            pltpu.VMEM((LANES // 2, LANES), jnp.float32),
            pltpu.SemaphoreType.DMA,
            pltpu.SemaphoreType.DMA,
        ],
        compiler_params=pltpu.CompilerParams(
            use_tc_tiling_on_sc=True, needs_layout_passes=False),
    )
    tail_pairs = X[VOCAB - D:].reshape(D // 2, LANES)
    x_pairs = transpose(X.T, tail_pairs)
    x_rm = x_pairs.reshape(VOCAB, D)

    gather = pl.kernel(
        functools.partial(_gather_body, b_per_w=b_per_w, nchunk=nchunk),
        mesh=mesh,
        out_type=jax.ShapeDtypeStruct((b_total, D), jnp.float32),
        scratch_types=(
            [pltpu.VMEM((b_per_w,), jnp.int32)]
            + [pltpu.VMEM((CHUNK, D), jnp.float32) for _ in range(NBUF)]
            + [pltpu.SemaphoreType.DMA for _ in range(2 * NBUF)]
        ),
        compiler_params=pltpu.CompilerParams(use_tc_tiling_on_sc=False),
    )
    out = gather(flat_idx, x_rm)
    return out.reshape(batch, n_fields, D)


# speedup vs baseline: 2.5100x; 2.5100x over previous
import functools
import jax
import jax.numpy as jnp
from jax import lax
from jax.experimental import pallas as pl
from jax.experimental.pallas import tpu as pltpu
from jax.experimental.pallas import tpu_sc as plsc

VOCAB = 1000000
D = 64
DP = 128
NC, NS = 2, 16
NW = NC * NS
CHUNK = 256
NBUF = 3

TVB = 2048
TGRID = -(-VOCAB // TVB)


def _tc_transpose_body(xt_ref, out_ref):
    out_ref[:, 0:D] = xt_ref[...].T


def _gather_body(idx_hbm, table_hbm, out_hbm, idx_v, *scratch,
                 b_per_w, nchunk):
    bufs = scratch[:NBUF]
    sem_g = scratch[NBUF:2 * NBUF]
    sem_s = scratch[2 * NBUF:3 * NBUF]

    wid = lax.axis_index("s") * NC + lax.axis_index("c")
    base = wid * b_per_w
    pltpu.sync_copy(idx_hbm.at[pl.ds(base, b_per_w)], idx_v)

    gathers = {}
    stores = {}
    store_waited = set()

    def start_gather(g):
        b = g % NBUF
        gathers[g] = pltpu.async_copy(
            table_hbm.at[idx_v.at[pl.ds(g * CHUNK, CHUNK)]], bufs[b],
            sem_g[b])

    for g in range(min(NBUF, nchunk)):
        start_gather(g)

    for g in range(nchunk):
        b = g % NBUF
        gathers[g].wait()
        stores[g] = pltpu.async_copy(
            bufs[b].at[:, pl.ds(0, D)],
            out_hbm.at[pl.ds(base + g * CHUNK, CHUNK)], sem_s[b])
        t = g - 1 + NBUF
        if g >= 1 and t < nchunk:
            stores[g - 1].wait()
            store_waited.add(g - 1)
            start_gather(t)

    for g in range(nchunk):
        if g not in store_waited:
            stores[g].wait()


def kernel(indices, X):
    batch, n_fields = indices.shape
    b_total = batch * n_fields
    assert b_total % (8 * NW) == 0
    b_per_w = b_total // NW
    assert b_per_w % CHUNK == 0
    nchunk = b_per_w // CHUNK

    flat_idx = indices.reshape(b_total).astype(jnp.int32)

    transpose = pl.pallas_call(
        _tc_transpose_body,
        grid=(TGRID,),
        in_specs=[pl.BlockSpec((D, TVB), lambda i: (0, i))],
        out_specs=pl.BlockSpec((TVB, DP), lambda i: (i, 0)),
        out_shape=jax.ShapeDtypeStruct((VOCAB, DP), jnp.float32),
    )
    x_wide = transpose(X.T)

    mesh = plsc.VectorSubcoreMesh(core_axis_name="c", subcore_axis_name="s")
    gather = pl.kernel(
        functools.partial(_gather_body, b_per_w=b_per_w, nchunk=nchunk),
        mesh=mesh,
        out_type=jax.ShapeDtypeStruct((b_total, D), jnp.float32),
        scratch_types=(
            [pltpu.VMEM((b_per_w,), jnp.int32)]
            + [pltpu.VMEM((CHUNK, DP), jnp.float32) for _ in range(NBUF)]
            + [pltpu.SemaphoreType.DMA for _ in range(2 * NBUF)]
        ),
        compiler_params=pltpu.CompilerParams(use_tc_tiling_on_sc=False),
    )
    out = gather(flat_idx, x_wide)
    return out.reshape(batch, n_fields, D)
